# Initial kernel scaffold; baseline (speedup 1.0000x reference)
import jax
import jax.numpy as jnp
from jax.experimental import pallas as pl


def kernel(x, bond_idx, angle_idx, torsion_idx, bond_repr, angle_repr, torsion_repr, bond_params, angle_params, torsion_params):
    raise NotImplementedError("write your pallas kernel here")


# speedup vs baseline: 1.8414x; 1.8414x over previous
import functools

import jax
import jax.numpy as jnp
from jax import lax
from jax.experimental import pallas as pl
from jax.experimental.pallas import tpu as pltpu
from jax.experimental.pallas import tpu_sc as plsc

D = 128
H = 64
C = 128
M = 4
NW = 32
LANES = 16


def _proj_body(x_ref, w_ref, tb_ref, ta2_ref, tta_ref, ttb_ref, tam_ref):
    res = jnp.dot(x_ref[...], w_ref[...], preferred_element_type=jnp.float32)
    tb_ref[...] = res[:, 0:128]
    ta2_ref[...] = res[:, 128:256]
    tta_ref[...] = res[:, 256:384]
    ttb_ref[...] = res[:, 384:512]
    tam_ref[...] = res[:, 512:640]


def _project(x, wproj):
    n_atoms = x.shape[0]
    blk = 1000
    grid = n_atoms // blk
    out_shapes = [
        jax.ShapeDtypeStruct((n_atoms, 128), jnp.float32),
        jax.ShapeDtypeStruct((n_atoms, 128), jnp.float32),
        jax.ShapeDtypeStruct((n_atoms, 128), jnp.float32),
        jax.ShapeDtypeStruct((n_atoms, 128), jnp.float32),
        jax.ShapeDtypeStruct((n_atoms, 128), jnp.float32),
    ]
    out_specs = [
        pl.BlockSpec((blk, 128), lambda i: (i, 0)),
        pl.BlockSpec((blk, 128), lambda i: (i, 0)),
        pl.BlockSpec((blk, 128), lambda i: (i, 0)),
        pl.BlockSpec((blk, 128), lambda i: (i, 0)),
        pl.BlockSpec((blk, 128), lambda i: (i, 0)),
    ]
    return pl.pallas_call(
        _proj_body,
        grid=(grid,),
        in_specs=[
            pl.BlockSpec((blk, D), lambda i: (i, 0)),
            pl.BlockSpec((D, 640), lambda i: (0, 0)),
        ],
        out_specs=out_specs,
        out_shape=out_shapes,
    )(x, wproj)


def _sc_gather_body(tb, ta2, tta, ttb, tam,
                    bidx, aidx, tidx,
                    zb, za, zt,
                    ibuf, gbuf, zbuf, sem):
    cid = lax.axis_index("c")
    sid = lax.axis_index("s")
    wid = sid * 2 + cid

    def do_type(idx_hbm, z_hbm, n_chunks, full):
        k = len(full)
        npad = n_chunks * C
        start = (wid * n_chunks) // NW
        end = ((wid + 1) * n_chunks) // NW

        def chunk(g, carry):
            base = g * C
            for p in range(k):
                pltpu.sync_copy(idx_hbm.at[pl.ds(p * npad + base, C)],
                                ibuf.at[p])
            waits = []
            for slot, (p, tref, _hf) in enumerate(full):
                waits.append(
                    pltpu.async_copy(tref.at[ibuf.at[p]], gbuf.at[slot], sem))
            for w in waits:
                w.wait()

            def acc_row(r, carry2):
                for j in range(H // LANES):
                    f = None
                    rv = None
                    for slot, (_p, _tref, hf) in enumerate(full):
                        gf = gbuf[slot, r, pl.ds(hf * H + j * LANES, LANES)]
                        gr = gbuf[slot, r,
                                  pl.ds((1 - hf) * H + j * LANES, LANES)]
                        f = gf if f is None else f + gf
                        rv = gr if rv is None else rv + gr
                    zbuf[0, r, pl.ds(j * LANES, LANES)] = f
                    zbuf[1, r, pl.ds(j * LANES, LANES)] = rv
                return carry2

            lax.fori_loop(0, C, acc_row, 0)
            pltpu.sync_copy(zbuf, z_hbm.at[g])
            return carry

        lax.fori_loop(start, end, chunk, 0)

    do_type(bidx, zb, zb.shape[0], [(0, tb, 0), (1, tb, 1)])
    do_type(aidx, za, za.shape[0], [(0, ta2, 0), (1, tam, 0), (2, ta2, 1)])
    do_type(tidx, zt, zt.shape[0],
            [(0, tta, 0), (1, ttb, 0), (2, ttb, 1), (3, tta, 1)])


def _sc_gather(tables, bidx, aidx, tidx, ncb, nca, nct):
    mesh = plsc.VectorSubcoreMesh(core_axis_name="c", subcore_axis_name="s")
    out_type = [
        jax.ShapeDtypeStruct((ncb, 2, C, H), jnp.float32),
        jax.ShapeDtypeStruct((nca, 2, C, H), jnp.float32),
        jax.ShapeDtypeStruct((nct, 2, C, H), jnp.float32),
    ]
    scratch = [
        pltpu.VMEM((4, C), jnp.int32),
        pltpu.VMEM((4, C, 128), jnp.float32),
        pltpu.VMEM((2, C, H), jnp.float32),
        pltpu.SemaphoreType.DMA,
    ]
    fn = pl.kernel(_sc_gather_body, out_type=out_type, mesh=mesh,
                   scratch_types=scratch)
    return fn(*tables, bidx, aidx, tidx)


def _tail_body(z_ref, r_ref, wr_ref, b1_ref, w2_ref, b2_ref, w3_ref, b3_ref,
               o_ref):
    zf = z_ref[:, 0].reshape(M * C, H)
    zr = z_ref[:, 1].reshape(M * C, H)
    base = r_ref[...] * wr_ref[...] + b1_ref[...]
    h1f = jnp.maximum(zf + base, 0.0)
    h1r = jnp.maximum(zr + base, 0.0)
    w2 = w2_ref[...]
    h2f = jnp.maximum(
        jnp.dot(h1f, w2, preferred_element_type=jnp.float32) + b2_ref[...], 0.0)
    h2r = jnp.maximum(
        jnp.dot(h1r, w2, preferred_element_type=jnp.float32) + b2_ref[...], 0.0)
    o_ref[...] = (jnp.dot(h2f + h2r, w3_ref[...],
                          preferred_element_type=jnp.float32) + b3_ref[...])


def _tail(z4, repr_pad, wr, b1, w2, b2, w3, b3):
    nch = z4.shape[0]
    grid = nch // M
    n_out = w3.shape[1]
    return pl.pallas_call(
        _tail_body,
        grid=(grid,),
        in_specs=[
            pl.BlockSpec((M, 2, C, H), lambda i: (i, 0, 0, 0)),
            pl.BlockSpec((M * C, 1), lambda i: (i, 0)),
            pl.BlockSpec((1, H), lambda i: (0, 0)),
            pl.BlockSpec((1, H), lambda i: (0, 0)),
            pl.BlockSpec((H, H), lambda i: (0, 0)),
            pl.BlockSpec((1, H), lambda i: (0, 0)),
            pl.BlockSpec((H, n_out), lambda i: (0, 0)),
            pl.BlockSpec((1, n_out), lambda i: (0, 0)),
        ],
        out_specs=pl.BlockSpec((M * C, n_out), lambda i: (i, 0)),
        out_shape=jax.ShapeDtypeStruct((nch * C, n_out), jnp.float32),
    )(z4, repr_pad, wr, b1, w2, b2, w3, b3)


def _pad_idx(idx, npad):
    n, k = idx.shape
    idx = idx.astype(jnp.int32)
    idxp = jnp.pad(idx, ((0, npad - n), (0, 0)))
    return idxp.T.reshape(-1)

def _pad_repr(r, npad):
    return jnp.pad(r, ((0, npad - r.shape[0]), (0, 0)))


def kernel(x, bond_idx, angle_idx, torsion_idx, bond_repr, angle_repr,
           torsion_repr, bond_params, angle_params, torsion_params):
    wb1, bb1, wb2, bb2, wb3, bb3 = bond_params
    wa1, ba1, wa2, ba2, wa3, ba3 = angle_params
    wt1, bt1, wt2, bt2, wt3, bt3 = torsion_params

    wproj = jnp.concatenate([
        wb1[0:D], wb1[D:2 * D],
        wa1[0:D], wa1[2 * D:3 * D],
        wt1[0:D], wt1[3 * D:4 * D],
        wt1[D:2 * D], wt1[2 * D:3 * D],
        wa1[D:2 * D], wa1[D:2 * D],
    ], axis=1)

    tables = _project(x, wproj)

    nb, na, nt = bond_idx.shape[0], angle_idx.shape[0], torsion_idx.shape[0]
    step = C * M
    ncb = ((nb + step - 1) // step) * M
    nca = ((na + step - 1) // step) * M
    nct = ((nt + step - 1) // step) * M

    bidx = _pad_idx(bond_idx, ncb * C)
    aidx = _pad_idx(angle_idx, nca * C)
    tidx = _pad_idx(torsion_idx, nct * C)

    zb, za, zt = _sc_gather(tables, bidx, aidx, tidx, ncb, nca, nct)

    def tail_for(z4, repr_, params):
        w1, b1, w2, b2, w3, b3 = params
        wr = w1[-1:, :]
        rp = _pad_repr(repr_, z4.shape[0] * C)
        return _tail(z4, rp, wr, b1.reshape(1, H), w2, b2.reshape(1, H),
                     w3, (2.0 * b3).reshape(1, -1))

    ob = tail_for(zb, bond_repr, bond_params)
    oa = tail_for(za, angle_repr, angle_params)
    ot = tail_for(zt, torsion_repr, torsion_params)

    return jnp.concatenate([ob[:nb], oa[:na], ot[:nt]], axis=0)


# speedup vs baseline: 1.9092x; 1.0368x over previous
import functools

import jax
import jax.numpy as jnp
from jax import lax
from jax.experimental import pallas as pl
from jax.experimental.pallas import tpu as pltpu
from jax.experimental.pallas import tpu_sc as plsc

D = 128
H = 64
C = 128
M = 4
NW = 32
LANES = 16


def _proj_body(x_ref, w_ref, tb_ref, ta2_ref, tta_ref, ttb_ref, tam_ref):
    res = jnp.dot(x_ref[...], w_ref[...], preferred_element_type=jnp.float32)
    tb_ref[...] = res[:, 0:128]
    ta2_ref[...] = res[:, 128:256]
    tta_ref[...] = res[:, 256:384]
    ttb_ref[...] = res[:, 384:512]
    tam_ref[...] = res[:, 512:640]


def _project(x, wproj):
    n_atoms = x.shape[0]
    blk = 1000
    grid = n_atoms // blk
    out_shapes = [
        jax.ShapeDtypeStruct((n_atoms, 128), jnp.float32),
        jax.ShapeDtypeStruct((n_atoms, 128), jnp.float32),
        jax.ShapeDtypeStruct((n_atoms, 128), jnp.float32),
        jax.ShapeDtypeStruct((n_atoms, 128), jnp.float32),
        jax.ShapeDtypeStruct((n_atoms, 128), jnp.float32),
    ]
    out_specs = [
        pl.BlockSpec((blk, 128), lambda i: (i, 0)),
        pl.BlockSpec((blk, 128), lambda i: (i, 0)),
        pl.BlockSpec((blk, 128), lambda i: (i, 0)),
        pl.BlockSpec((blk, 128), lambda i: (i, 0)),
        pl.BlockSpec((blk, 128), lambda i: (i, 0)),
    ]
    return pl.pallas_call(
        _proj_body,
        grid=(grid,),
        in_specs=[
            pl.BlockSpec((blk, D), lambda i: (i, 0)),
            pl.BlockSpec((D, 640), lambda i: (0, 0)),
        ],
        out_specs=out_specs,
        out_shape=out_shapes,
    )(x, wproj)


def _sc_gather_body(tb, ta2, tta, ttb, tam,
                    bidx, aidx, tidx,
                    zb, za, zt,
                    ibuf, gbuf, zbuf, sem):
    cid = lax.axis_index("c")
    sid = lax.axis_index("s")
    wid = sid * 2 + cid

    def do_type(idx_hbm, z_hbm, n_chunks, full):
        k = len(full)
        npad = n_chunks * C
        start = (wid * n_chunks) // NW
        end = ((wid + 1) * n_chunks) // NW

        def chunk(g, carry):
            base = g * C
            for p in range(k):
                pltpu.sync_copy(idx_hbm.at[pl.ds(p * npad + base, C)],
                                ibuf.at[p])
            waits = []
            for slot, (p, tref, _hf) in enumerate(full):
                waits.append(
                    pltpu.async_copy(tref.at[ibuf.at[p]], gbuf.at[slot], sem))
            for w in waits:
                w.wait()

            def acc_row(r, carry2):
                for j in range(H // LANES):
                    f = None
                    rv = None
                    for slot, (_p, _tref, hf) in enumerate(full):
                        gf = gbuf[slot, r, pl.ds(hf * H + j * LANES, LANES)]
                        gr = gbuf[slot, r,
                                  pl.ds((1 - hf) * H + j * LANES, LANES)]
                        f = gf if f is None else f + gf
                        rv = gr if rv is None else rv + gr
                    zbuf[r, pl.ds(j * LANES, LANES)] = f
                    zbuf[r, pl.ds(H + j * LANES, LANES)] = rv
                return carry2

            lax.fori_loop(0, C, acc_row, 0)
            pltpu.sync_copy(zbuf, z_hbm.at[g])
            return carry

        lax.fori_loop(start, end, chunk, 0)

    do_type(bidx, zb, zb.shape[0], [(0, tb, 0), (1, tb, 1)])
    do_type(aidx, za, za.shape[0], [(0, ta2, 0), (1, tam, 0), (2, ta2, 1)])
    do_type(tidx, zt, zt.shape[0],
            [(0, tta, 0), (1, ttb, 0), (2, ttb, 1), (3, tta, 1)])


def _sc_gather(tables, bidx, aidx, tidx, ncb, nca, nct):
    mesh = plsc.VectorSubcoreMesh(core_axis_name="c", subcore_axis_name="s")
    out_type = [
        jax.ShapeDtypeStruct((ncb, C, 2 * H), jnp.float32),
        jax.ShapeDtypeStruct((nca, C, 2 * H), jnp.float32),
        jax.ShapeDtypeStruct((nct, C, 2 * H), jnp.float32),
    ]
    scratch = [
        pltpu.VMEM((4, C), jnp.int32),
        pltpu.VMEM((4, C, 128), jnp.float32),
        pltpu.VMEM((C, 2 * H), jnp.float32),
        pltpu.SemaphoreType.DMA,
    ]
    fn = pl.kernel(_sc_gather_body, out_type=out_type, mesh=mesh,
                   scratch_types=scratch)
    return fn(*tables, bidx, aidx, tidx)


def _tail_body(z_ref, r_ref, wr_ref, b1_ref, w2_ref, b2_ref, w3_ref, b3_ref,
               o_ref):
    z = z_ref[...].reshape(M * C, 2 * H)
    zf = z[:, 0:H]
    zr = z[:, H:2 * H]
    base = r_ref[...] * wr_ref[...] + b1_ref[...]
    h1f = jnp.maximum(zf + base, 0.0)
    h1r = jnp.maximum(zr + base, 0.0)
    w2 = w2_ref[...]
    h2f = jnp.maximum(
        jnp.dot(h1f, w2, preferred_element_type=jnp.float32) + b2_ref[...], 0.0)
    h2r = jnp.maximum(
        jnp.dot(h1r, w2, preferred_element_type=jnp.float32) + b2_ref[...], 0.0)
    o_ref[...] = (jnp.dot(h2f + h2r, w3_ref[...],
                          preferred_element_type=jnp.float32) + b3_ref[...])


def _tail(z4, repr_pad, wr, b1, w2, b2, w3, b3):
    nch = z4.shape[0]
    grid = nch // M
    n_out = w3.shape[1]
    return pl.pallas_call(
        _tail_body,
        grid=(grid,),
        in_specs=[
            pl.BlockSpec((M, C, 2 * H), lambda i: (i, 0, 0)),
            pl.BlockSpec((M * C, 1), lambda i: (i, 0)),
            pl.BlockSpec((1, H), lambda i: (0, 0)),
            pl.BlockSpec((1, H), lambda i: (0, 0)),
            pl.BlockSpec((H, H), lambda i: (0, 0)),
            pl.BlockSpec((1, H), lambda i: (0, 0)),
            pl.BlockSpec((H, n_out), lambda i: (0, 0)),
            pl.BlockSpec((1, n_out), lambda i: (0, 0)),
        ],
        out_specs=pl.BlockSpec((M * C, n_out), lambda i: (i, 0)),
        out_shape=jax.ShapeDtypeStruct((nch * C, n_out), jnp.float32),
    )(z4, repr_pad, wr, b1, w2, b2, w3, b3)


def _pad_idx(idx, npad):
    n, k = idx.shape
    idx = idx.astype(jnp.int32)
    idxp = jnp.pad(idx, ((0, npad - n), (0, 0)))
    return idxp.T.reshape(-1)

def _pad_repr(r, npad):
    return jnp.pad(r, ((0, npad - r.shape[0]), (0, 0)))


def kernel(x, bond_idx, angle_idx, torsion_idx, bond_repr, angle_repr,
           torsion_repr, bond_params, angle_params, torsion_params):
    wb1, bb1, wb2, bb2, wb3, bb3 = bond_params
    wa1, ba1, wa2, ba2, wa3, ba3 = angle_params
    wt1, bt1, wt2, bt2, wt3, bt3 = torsion_params

    wproj = jnp.concatenate([
        wb1[0:D], wb1[D:2 * D],
        wa1[0:D], wa1[2 * D:3 * D],
        wt1[0:D], wt1[3 * D:4 * D],
        wt1[D:2 * D], wt1[2 * D:3 * D],
        wa1[D:2 * D], wa1[D:2 * D],
    ], axis=1)

    tables = _project(x, wproj)

    nb, na, nt = bond_idx.shape[0], angle_idx.shape[0], torsion_idx.shape[0]
    step = C * M
    ncb = ((nb + step - 1) // step) * M
    nca = ((na + step - 1) // step) * M
    nct = ((nt + step - 1) // step) * M

    bidx = _pad_idx(bond_idx, ncb * C)
    aidx = _pad_idx(angle_idx, nca * C)
    tidx = _pad_idx(torsion_idx, nct * C)

    zb, za, zt = _sc_gather(tables, bidx, aidx, tidx, ncb, nca, nct)

    def tail_for(z4, repr_, params):
        w1, b1, w2, b2, w3, b3 = params
        wr = w1[-1:, :]
        rp = _pad_repr(repr_, z4.shape[0] * C)
        return _tail(z4, rp, wr, b1.reshape(1, H), w2, b2.reshape(1, H),
                     w3, (2.0 * b3).reshape(1, -1))

    ob = tail_for(zb, bond_repr, bond_params)
    oa = tail_for(za, angle_repr, angle_params)
    ot = tail_for(zt, torsion_repr, torsion_params)

    return jnp.concatenate([ob[:nb], oa[:na], ot[:nt]], axis=0)
